# baseline (speedup 1.0000x reference)
import functools

import jax
import jax.numpy as jnp
from jax import lax
from jax.experimental import pallas as pl
from jax.experimental.pallas import tpu as pltpu
from jax.experimental.pallas import tpu_sc as plsc

N = 10000
F = 128
CH = 128
NC = 40
E = 320000
EPS = 1e-05

NP = 10240
RPT = NP // 16
CHUNK = 128
NTILES = 32
NCH = 2 * (-(-E // (NTILES * CHUNK * 2)))
EP = NTILES * NCH * CHUNK
BN_C = (1.0 + EPS) ** -0.5

_MESH = plsc.VectorSubcoreMesh(core_axis_name="c", subcore_axis_name="s")


def _make_prop(D):
    @functools.partial(
        pl.kernel,
        mesh=_MESH,
        out_type=jax.ShapeDtypeStruct((2, NP, D), jnp.float32),
        scratch_types=[
            pltpu.VMEM((NCH, CHUNK), jnp.int32),
            pltpu.VMEM((CHUNK,), jnp.int32),
            pltpu.VMEM((CHUNK,), jnp.int32),
            pltpu.VMEM((CHUNK,), jnp.int32),
            pltpu.VMEM((CHUNK,), jnp.int32),
            pltpu.VMEM((CHUNK, D), jnp.float32),
            pltpu.VMEM((CHUNK, D), jnp.float32),
            pltpu.VMEM_SHARED((NP, D), jnp.float32),
            pltpu.SemaphoreType.DMA,
            pltpu.SemaphoreType.DMA,
            pltpu.SemaphoreType.DMA,
        ],
    )
    def prop_k(xs, pkb, zrows, out, pk, s0, s1, d0, d1, b0, b1, acc,
               sem0, sem1, semz):
        cid = lax.axis_index("c")
        sid = lax.axis_index("s")
        r0 = sid * RPT
        pltpu.sync_copy(pkb.at[cid, sid], pk)
        zc = pltpu.async_copy(zrows.at[pl.ds(r0, RPT)],
                              acc.at[pl.ds(r0, RPT)], semz)

        def unpack(j, sidx, didx):
            for k in range(CHUNK // 16):
                v = pk[j, pl.ds(16 * k, 16)]
                sidx[pl.ds(16 * k, 16)] = v & 0xFFFF
                didx[pl.ds(16 * k, 16)] = lax.shift_right_logical(v, 16)

        def gather(buf, sem, sidx):
            return pltpu.async_copy(xs.at[sidx], buf, sem)

        unpack(0, s0, d0)
        gather(b0, sem0, s0)
        zc.wait()
        plsc.subcore_barrier()

        def step(i, carry):
            j0 = 2 * i
            unpack(j0 + 1, s1, d1)
            gather(b1, sem1, s1)
            pltpu.make_async_copy(xs.at[s0], b0, sem0).wait()
            pltpu.sync_copy(b0, acc.at[d0], add=True)
            unpack(j0 + 2, s0, d0)
            gather(b0, sem0, s0)
            pltpu.make_async_copy(xs.at[s1], b1, sem1).wait()
            pltpu.sync_copy(b1, acc.at[d1], add=True)
            return carry

        lax.fori_loop(0, NCH // 2 - 1, step, 0)
        unpack(NCH - 1, s1, d1)
        gather(b1, sem1, s1)
        pltpu.make_async_copy(xs.at[s0], b0, sem0).wait()
        pltpu.sync_copy(b0, acc.at[d0], add=True)
        pltpu.make_async_copy(xs.at[s1], b1, sem1).wait()
        pltpu.sync_copy(b1, acc.at[d1], add=True)

        plsc.subcore_barrier()
        pltpu.sync_copy(acc.at[pl.ds(r0, RPT)], out.at[cid, pl.ds(r0, RPT)])

    return prop_k


_prop128 = _make_prop(128)


@functools.partial(
    pl.kernel,
    mesh=_MESH,
    out_type=jax.ShapeDtypeStruct((2, NP), jnp.float32),
    scratch_types=[
        pltpu.VMEM((NCH, CHUNK), jnp.int32),
        pltpu.VMEM((CHUNK,), jnp.float32),
        pltpu.VMEM_SHARED((NP,), jnp.float32),
    ],
)
def _deg_k(dstb, z1d, out, idx_d, ones_v, acc):
    cid = lax.axis_index("c")
    sid = lax.axis_index("s")
    r0 = sid * RPT
    pltpu.sync_copy(z1d.at[pl.ds(r0, RPT)], acc.at[pl.ds(r0, RPT)])
    for k in range(CHUNK // 16):
        ones_v[pl.ds(k * 16, 16)] = jnp.ones((16,), jnp.float32)
    pltpu.sync_copy(dstb.at[cid, sid], idx_d)
    plsc.subcore_barrier()

    def step(j, carry):
        pltpu.sync_copy(ones_v, acc.at[idx_d.at[j]], add=True)
        return carry

    lax.fori_loop(0, NCH, step, 0)
    plsc.subcore_barrier()
    pltpu.sync_copy(acc.at[pl.ds(r0, RPT)], out.at[cid, pl.ds(r0, RPT)])


_BN = NP // 16


def _dinv_of(cnt_ref):
    cnt = cnt_ref[:, 0:1] + cnt_ref[:, 1:2]
    return lax.rsqrt(cnt + 1.0)


def _k0_body(h_ref, w_ref, cnt_ref, o_ref):
    y = jnp.dot(h_ref[...], w_ref[...], preferred_element_type=jnp.float32)
    o_ref[...] = y * _dinv_of(cnt_ref)


def _kmid_body(acc_ref, ys_ref, cnt_ref, w_ref, o_ref):
    dinv = _dinv_of(cnt_ref)
    s = acc_ref[0] + acc_ref[1] + ys_ref[...]
    t = jnp.maximum(BN_C * dinv * s, 0.0)
    o_ref[...] = jnp.dot(t, w_ref[...],
                         preferred_element_type=jnp.float32) * dinv


def _kact_body(acc_ref, ys_ref, cnt_ref, o_ref):
    dinv = _dinv_of(cnt_ref)
    s = acc_ref[0] + acc_ref[1] + ys_ref[...]
    o_ref[...] = jnp.maximum(BN_C * dinv * s, 0.0) * dinv


def _kend_body(acc_ref, ys_ref, cnt_ref, w_ref, b_ref, o_ref):
    dinv = _dinv_of(cnt_ref)
    s = acc_ref[0] + acc_ref[1] + ys_ref[...]
    o_ref[...] = jnp.dot(dinv * s, w_ref[...],
                         preferred_element_type=jnp.float32) + b_ref[...]


def _cnt_spec():
    return pl.BlockSpec((_BN, 2), lambda i: (i, 0))


def _tc_k0(h, w, cnt2):
    return pl.pallas_call(
        _k0_body,
        grid=(NP // _BN,),
        in_specs=[pl.BlockSpec((_BN, F), lambda i: (i, 0)),
                  pl.BlockSpec((F, CH), lambda i: (0, 0)),
                  _cnt_spec()],
        out_specs=pl.BlockSpec((_BN, CH), lambda i: (i, 0)),
        out_shape=jax.ShapeDtypeStruct((NP, CH), jnp.float32),
    )(h, w, cnt2)


def _tc_kmid(acc2, ys, cnt2, w):
    d_in = ys.shape[1]
    d_out = w.shape[1]
    return pl.pallas_call(
        _kmid_body,
        grid=(NP // _BN,),
        in_specs=[pl.BlockSpec((2, _BN, d_in), lambda i: (0, i, 0)),
                  pl.BlockSpec((_BN, d_in), lambda i: (i, 0)),
                  _cnt_spec(),
                  pl.BlockSpec((d_in, d_out), lambda i: (0, 0))],
        out_specs=pl.BlockSpec((_BN, d_out), lambda i: (i, 0)),
        out_shape=jax.ShapeDtypeStruct((NP, d_out), jnp.float32),
    )(acc2, ys, cnt2, w)


def _tc_kact(acc2, ys, cnt2):
    return pl.pallas_call(
        _kact_body,
        grid=(NP // _BN,),
        in_specs=[pl.BlockSpec((2, _BN, CH), lambda i: (0, i, 0)),
                  pl.BlockSpec((_BN, CH), lambda i: (i, 0)),
                  _cnt_spec()],
        out_specs=pl.BlockSpec((_BN, CH), lambda i: (i, 0)),
        out_shape=jax.ShapeDtypeStruct((NP, CH), jnp.float32),
    )(acc2, ys, cnt2)


def _tc_kend(acc2, ys, cnt2, w, b):
    return pl.pallas_call(
        _kend_body,
        grid=(NP // _BN,),
        in_specs=[pl.BlockSpec((2, _BN, CH), lambda i: (0, i, 0)),
                  pl.BlockSpec((_BN, CH), lambda i: (i, 0)),
                  _cnt_spec(),
                  pl.BlockSpec((CH, NC), lambda i: (0, 0)),
                  pl.BlockSpec((1, NC), lambda i: (0, 0))],
        out_specs=pl.BlockSpec((_BN, NC), lambda i: (i, 0)),
        out_shape=jax.ShapeDtypeStruct((NP, NC), jnp.float32),
    )(acc2, ys, cnt2, w, b)


def kernel(h, edge_index, W1, W2, W3, b3):
    pad_ids = N + (jnp.arange(EP - E, dtype=jnp.int32) % (NP - N))
    src_p = jnp.concatenate([edge_index[0], pad_ids])
    dst_p = jnp.concatenate([edge_index[1], pad_ids])
    pkb = (src_p | (dst_p << 16)).reshape(2, 16, NCH, CHUNK)
    dstb = dst_p.reshape(2, 16, NCH, CHUNK)
    h_pad = jnp.pad(h, ((0, NP - N), (0, 0)))
    b3_2d = b3.reshape(1, NC)
    zrows = jnp.zeros((NP, CH), jnp.float32)
    z1d = jnp.zeros((NP,), jnp.float32)

    cnt2 = _deg_k(dstb, z1d).T
    ys1 = _tc_k0(h_pad, W1, cnt2)

    acc1 = _prop128(ys1, pkb, zrows)
    ys2 = _tc_kmid(acc1, ys1, cnt2, W2)
    acc2 = _prop128(ys2, pkb, zrows)
    ys3 = _tc_kact(acc2, ys2, cnt2)
    acc3 = _prop128(ys3, pkb, zrows)
    out = _tc_kend(acc3, ys3, cnt2, W3, b3_2d)
    return out[:N]


# speedup vs baseline: 1.0044x; 1.0044x over previous
import functools

import jax
import jax.numpy as jnp
from jax import lax
from jax.experimental import pallas as pl
from jax.experimental.pallas import tpu as pltpu
from jax.experimental.pallas import tpu_sc as plsc

N = 10000
F = 128
CH = 128
NC = 40
E = 320000
EPS = 1e-05

NP = 10240
RPT = NP // 16
CHUNK = 128
NTILES = 32
NCH = -(-E // (NTILES * CHUNK))
EP = NTILES * NCH * CHUNK
BN_C = (1.0 + EPS) ** -0.5

_MESH = plsc.VectorSubcoreMesh(core_axis_name="c", subcore_axis_name="s")


def _make_prop(D):
    @functools.partial(
        pl.kernel,
        mesh=_MESH,
        out_type=jax.ShapeDtypeStruct((2, NP, D), jnp.float32),
        scratch_types=[
            pltpu.VMEM((NCH, CHUNK), jnp.int32),
            pltpu.VMEM((CHUNK,), jnp.int32),
            pltpu.VMEM((CHUNK,), jnp.int32),
            pltpu.VMEM((CHUNK,), jnp.int32),
            pltpu.VMEM((CHUNK,), jnp.int32),
            pltpu.VMEM((CHUNK, D), jnp.float32),
            pltpu.VMEM((CHUNK, D), jnp.float32),
            pltpu.VMEM_SHARED((NP, D), jnp.float32),
            pltpu.SemaphoreType.DMA,
            pltpu.SemaphoreType.DMA,
            pltpu.SemaphoreType.DMA,
        ],
    )
    def prop_k(xs, pkb, zrows, out, pk, s0, s1, d0, d1, b0, b1, acc,
               sem0, sem1, semz):
        cid = lax.axis_index("c")
        sid = lax.axis_index("s")
        r0 = sid * RPT
        pltpu.sync_copy(pkb.at[cid, sid], pk)
        zc = pltpu.async_copy(zrows.at[pl.ds(r0, RPT)],
                              acc.at[pl.ds(r0, RPT)], semz)

        def unpack(j, sidx, didx):
            for k in range(CHUNK // 16):
                v = pk[j, pl.ds(16 * k, 16)]
                sidx[pl.ds(16 * k, 16)] = v & 0xFFFF
                didx[pl.ds(16 * k, 16)] = lax.shift_right_logical(v, 16)

        def gather(buf, sem, sidx):
            return pltpu.async_copy(xs.at[sidx], buf, sem)

        unpack(0, s0, d0)
        gather(b0, sem0, s0)
        zc.wait()
        plsc.subcore_barrier()

        def step(i, carry):
            j0 = 2 * i
            unpack(j0 + 1, s1, d1)
            gather(b1, sem1, s1)
            pltpu.make_async_copy(xs.at[s0], b0, sem0).wait()
            pltpu.sync_copy(b0, acc.at[d0], add=True)
            unpack(j0 + 2, s0, d0)
            gather(b0, sem0, s0)
            pltpu.make_async_copy(xs.at[s1], b1, sem1).wait()
            pltpu.sync_copy(b1, acc.at[d1], add=True)
            return carry

        lax.fori_loop(0, NCH // 2, step, 0)
        pltpu.make_async_copy(xs.at[s0], b0, sem0).wait()
        pltpu.sync_copy(b0, acc.at[d0], add=True)

        plsc.subcore_barrier()
        pltpu.sync_copy(acc.at[pl.ds(r0, RPT)], out.at[cid, pl.ds(r0, RPT)])

    return prop_k


_prop128 = _make_prop(128)


@functools.partial(
    pl.kernel,
    mesh=_MESH,
    out_type=jax.ShapeDtypeStruct((2, NP), jnp.float32),
    scratch_types=[
        pltpu.VMEM((NCH, CHUNK), jnp.int32),
        pltpu.VMEM((CHUNK,), jnp.float32),
        pltpu.VMEM_SHARED((NP,), jnp.float32),
    ],
)
def _deg_k(dstb, z1d, out, idx_d, ones_v, acc):
    cid = lax.axis_index("c")
    sid = lax.axis_index("s")
    r0 = sid * RPT
    pltpu.sync_copy(z1d.at[pl.ds(r0, RPT)], acc.at[pl.ds(r0, RPT)])
    for k in range(CHUNK // 16):
        ones_v[pl.ds(k * 16, 16)] = jnp.ones((16,), jnp.float32)
    pltpu.sync_copy(dstb.at[cid, sid], idx_d)
    plsc.subcore_barrier()

    def step(j, carry):
        pltpu.sync_copy(ones_v, acc.at[idx_d.at[j]], add=True)
        return carry

    lax.fori_loop(0, NCH, step, 0)
    plsc.subcore_barrier()
    pltpu.sync_copy(acc.at[pl.ds(r0, RPT)], out.at[cid, pl.ds(r0, RPT)])


_BN = NP // 16


def _dinv_of(cnt_ref):
    cnt = cnt_ref[:, 0:1] + cnt_ref[:, 1:2]
    return lax.rsqrt(cnt + 1.0)


def _k0_body(h_ref, w_ref, cnt_ref, o_ref):
    y = jnp.dot(h_ref[...], w_ref[...], preferred_element_type=jnp.float32)
    o_ref[...] = y * _dinv_of(cnt_ref)


def _kmid_body(acc_ref, ys_ref, cnt_ref, w_ref, o_ref):
    dinv = _dinv_of(cnt_ref)
    s = acc_ref[0] + acc_ref[1] + ys_ref[...]
    t = jnp.maximum(BN_C * dinv * s, 0.0)
    o_ref[...] = jnp.dot(t, w_ref[...],
                         preferred_element_type=jnp.float32) * dinv


def _kact_body(acc_ref, ys_ref, cnt_ref, o_ref):
    dinv = _dinv_of(cnt_ref)
    s = acc_ref[0] + acc_ref[1] + ys_ref[...]
    o_ref[...] = jnp.maximum(BN_C * dinv * s, 0.0) * dinv


def _kend_body(acc_ref, ys_ref, cnt_ref, w_ref, b_ref, o_ref):
    dinv = _dinv_of(cnt_ref)
    s = acc_ref[0] + acc_ref[1] + ys_ref[...]
    o_ref[...] = jnp.dot(dinv * s, w_ref[...],
                         preferred_element_type=jnp.float32) + b_ref[...]


def _cnt_spec():
    return pl.BlockSpec((_BN, 2), lambda i: (i, 0))


def _tc_k0(h, w, cnt2):
    return pl.pallas_call(
        _k0_body,
        grid=(NP // _BN,),
        in_specs=[pl.BlockSpec((_BN, F), lambda i: (i, 0)),
                  pl.BlockSpec((F, CH), lambda i: (0, 0)),
                  _cnt_spec()],
        out_specs=pl.BlockSpec((_BN, CH), lambda i: (i, 0)),
        out_shape=jax.ShapeDtypeStruct((NP, CH), jnp.float32),
    )(h, w, cnt2)


def _tc_kmid(acc2, ys, cnt2, w):
    d_in = ys.shape[1]
    d_out = w.shape[1]
    return pl.pallas_call(
        _kmid_body,
        grid=(NP // _BN,),
        in_specs=[pl.BlockSpec((2, _BN, d_in), lambda i: (0, i, 0)),
                  pl.BlockSpec((_BN, d_in), lambda i: (i, 0)),
                  _cnt_spec(),
                  pl.BlockSpec((d_in, d_out), lambda i: (0, 0))],
        out_specs=pl.BlockSpec((_BN, d_out), lambda i: (i, 0)),
        out_shape=jax.ShapeDtypeStruct((NP, d_out), jnp.float32),
    )(acc2, ys, cnt2, w)


def _tc_kact(acc2, ys, cnt2):
    return pl.pallas_call(
        _kact_body,
        grid=(NP // _BN,),
        in_specs=[pl.BlockSpec((2, _BN, CH), lambda i: (0, i, 0)),
                  pl.BlockSpec((_BN, CH), lambda i: (i, 0)),
                  _cnt_spec()],
        out_specs=pl.BlockSpec((_BN, CH), lambda i: (i, 0)),
        out_shape=jax.ShapeDtypeStruct((NP, CH), jnp.float32),
    )(acc2, ys, cnt2)


def _tc_kend(acc2, ys, cnt2, w, b):
    return pl.pallas_call(
        _kend_body,
        grid=(NP // _BN,),
        in_specs=[pl.BlockSpec((2, _BN, CH), lambda i: (0, i, 0)),
                  pl.BlockSpec((_BN, CH), lambda i: (i, 0)),
                  _cnt_spec(),
                  pl.BlockSpec((CH, NC), lambda i: (0, 0)),
                  pl.BlockSpec((1, NC), lambda i: (0, 0))],
        out_specs=pl.BlockSpec((_BN, NC), lambda i: (i, 0)),
        out_shape=jax.ShapeDtypeStruct((NP, NC), jnp.float32),
    )(acc2, ys, cnt2, w, b)


def kernel(h, edge_index, W1, W2, W3, b3):
    pad_ids = N + (jnp.arange(EP - E, dtype=jnp.int32) % (NP - N))
    src_p = jnp.concatenate([edge_index[0], pad_ids])
    dst_p = jnp.concatenate([edge_index[1], pad_ids])
    pkb = (src_p | (dst_p << 16)).reshape(2, 16, NCH, CHUNK)
    dstb = dst_p.reshape(2, 16, NCH, CHUNK)
    h_pad = jnp.pad(h, ((0, NP - N), (0, 0)))
    b3_2d = b3.reshape(1, NC)
    zrows = jnp.zeros((NP, CH), jnp.float32)
    z1d = jnp.zeros((NP,), jnp.float32)

    cnt2 = _deg_k(dstb, z1d).T
    ys1 = _tc_k0(h_pad, W1, cnt2)

    acc1 = _prop128(ys1, pkb, zrows)
    ys2 = _tc_kmid(acc1, ys1, cnt2, W2)
    acc2 = _prop128(ys2, pkb, zrows)
    ys3 = _tc_kact(acc2, ys2, cnt2)
    acc3 = _prop128(ys3, pkb, zrows)
    out = _tc_kend(acc3, ys3, cnt2, W3, b3_2d)
    return out[:N]
